# baseline (speedup 1.0000x reference)
import numpy as np
import jax
import jax.numpy as jnp
from jax import lax
from jax.experimental import pallas as pl
from jax.experimental.pallas import tpu as pltpu
from jax.experimental.pallas import tpu_sc as plsc

_B, _C, _L = 16, 192, 4096
_S = 64
_ROWS = _B * _C
_NC, _NS = 2, 16
_NW = _NC * _NS
_RPW = _ROWS // _NW
_LANES = 16
_G = 6
_GL = _G * _L
_NG = _RPW // _G


def _morton_idx(l: int) -> np.ndarray:
    s = int(np.sqrt(l))
    ij = np.arange(l, dtype=np.int64)
    i = np.zeros(l, dtype=np.int64)
    j = np.zeros(l, dtype=np.int64)
    for t in range(int(l).bit_length() // 2 + 1):
        i += ((ij >> (2 * t + 1)) & 1) << t
        j += ((ij >> (2 * t)) & 1) << t
    return (i * s + j).astype(np.int32)


_IDX_NP = _morton_idx(_L)


def _sc_body(x_hbm, idx_hbm, out_hbm, idx_v, in_v, out_v, in_sem, out_sem):
    wid = lax.axis_index("s") * _NC + lax.axis_index("c")
    row0 = wid * _RPW
    pltpu.sync_copy(idx_hbm, idx_v)

    def load(g, b):
        for r in range(_G):
            pltpu.async_copy(x_hbm.at[row0 + g * _G + r],
                             in_v.at[pl.ds((b * _G + r) * _L, _L)],
                             in_sem.at[b])

    def wait_in(b):
        for r in range(_G):
            pltpu.make_async_copy(x_hbm.at[0],
                                  in_v.at[pl.ds((b * _G + r) * _L, _L)],
                                  in_sem.at[b]).wait()

    def store(g, b):
        for r in range(_G):
            pltpu.async_copy(out_v.at[pl.ds((b * _G + r) * _L, _L)],
                             out_hbm.at[row0 + g * _G + r], out_sem.at[b])

    def wait_out(b):
        for r in range(_G):
            pltpu.make_async_copy(out_v.at[pl.ds((b * _G + r) * _L, _L)],
                                  out_hbm.at[0], out_sem.at[b]).wait()

    load(0, 0)

    @pl.loop(0, _NG, step=2)
    def _grp(g0):
        for b in range(2):
            g = g0 + b

            @pl.when(g + 1 < _NG)
            def _():
                load(g + 1, 1 - b)

            wait_in(b)

            @pl.when(g >= 2)
            def _():
                wait_out(b)

            for r in range(_G):
                o = (b * _G + r) * _L

                @plsc.parallel_loop(0, _L // _LANES, unroll=8)
                def _blk(k):
                    p = k * _LANES
                    v = in_v[pl.ds(o + p, _LANES)]
                    iv = idx_v[pl.ds(p, _LANES)] + o
                    plsc.store_scatter(out_v, [iv], v)

            store(g, b)

    wait_out(0)
    wait_out(1)


def _sc_permute(xf, idx):
    mesh = plsc.VectorSubcoreMesh(core_axis_name="c", subcore_axis_name="s")
    return pl.kernel(
        _sc_body,
        out_type=jax.ShapeDtypeStruct((_ROWS, _L), jnp.float32),
        mesh=mesh,
        scratch_types=[
            pltpu.VMEM((_L,), jnp.int32),
            pltpu.VMEM((2 * _GL,), jnp.float32),
            pltpu.VMEM((2 * _GL,), jnp.float32),
            pltpu.SemaphoreType.DMA((2,)),
            pltpu.SemaphoreType.DMA((2,)),
        ],
        compiler_params=pltpu.CompilerParams(needs_layout_passes=False),
    )(xf, idx)


_CH = 1024


def _tc_tr_body(z_ref, w_ref):
    zb = z_ref[0]
    r = lax.broadcasted_iota(jnp.int32, (_C, _C), 0)
    c = lax.broadcasted_iota(jnp.int32, (_C, _C), 1)
    eye = (r == c).astype(jnp.float32)
    w_ref[0] = lax.dot_general(zb, eye, (((0,), (0,)), ((), ())),
                               preferred_element_type=jnp.float32,
                               precision=lax.Precision.HIGHEST)


def _tc_transpose(z3):
    return pl.pallas_call(
        _tc_tr_body,
        out_shape=jax.ShapeDtypeStruct((_B, _L, _C), jnp.float32),
        grid=(_B, _L // _CH),
        in_specs=[pl.BlockSpec((1, _C, _CH), lambda b, k: (b, 0, k))],
        out_specs=pl.BlockSpec((1, _CH, _C), lambda b, k: (b, k, 0)),
        compiler_params=pltpu.CompilerParams(
            dimension_semantics=("parallel", "parallel")),
    )(z3)


def kernel(x):
    xf = x.reshape(_ROWS, _L)
    idx = jnp.asarray(_IDX_NP)
    z = _sc_permute(xf, idx)
    w = _tc_transpose(z.reshape(_B, _C, _L))
    y = w.reshape(_B, _S, _S, _C)
    return y.transpose(0, 3, 1, 2)


# speedup vs baseline: 1.3476x; 1.3476x over previous
import numpy as np
import jax
import jax.numpy as jnp
from jax import lax
from jax.experimental import pallas as pl
from jax.experimental.pallas import tpu as pltpu
from jax.experimental.pallas import tpu_sc as plsc

_B, _C, _L = 16, 192, 4096
_S = 64
_ROWS = _B * _C
_NC, _NS = 2, 16
_NW = _NC * _NS
_RPW = _ROWS // _NW
_LANES = 16
_G = 6
_GL = _G * _L
_NG = _RPW // _G


def _morton_idx(l: int) -> np.ndarray:
    s = int(np.sqrt(l))
    ij = np.arange(l, dtype=np.int64)
    i = np.zeros(l, dtype=np.int64)
    j = np.zeros(l, dtype=np.int64)
    for t in range(int(l).bit_length() // 2 + 1):
        i += ((ij >> (2 * t + 1)) & 1) << t
        j += ((ij >> (2 * t)) & 1) << t
    return (i * s + j).astype(np.int32)


_IDX_NP = _morton_idx(_L)


def _sc_body(x_hbm, idx_hbm, out_hbm, idx_v, in_v, out_v, in_sem, out_sem):
    wid = lax.axis_index("s") * _NC + lax.axis_index("c")
    row0 = wid * _RPW
    pltpu.sync_copy(idx_hbm, idx_v)

    def load(g, b):
        for r in range(_G):
            pltpu.async_copy(x_hbm.at[row0 + g * _G + r],
                             in_v.at[pl.ds((b * _G + r) * _L, _L)],
                             in_sem.at[b])

    def wait_in(b):
        for r in range(_G):
            pltpu.make_async_copy(x_hbm.at[0],
                                  in_v.at[pl.ds((b * _G + r) * _L, _L)],
                                  in_sem.at[b]).wait()

    def store(g, b):
        for r in range(_G):
            pltpu.async_copy(out_v.at[pl.ds((b * _G + r) * _L, _L)],
                             out_hbm.at[row0 + g * _G + r], out_sem.at[b])

    def wait_out(b):
        for r in range(_G):
            pltpu.make_async_copy(out_v.at[pl.ds((b * _G + r) * _L, _L)],
                                  out_hbm.at[0], out_sem.at[b]).wait()

    load(0, 0)

    @pl.loop(0, _NG, step=2)
    def _grp(g0):
        for b in range(2):
            g = g0 + b

            @pl.when(g + 1 < _NG)
            def _():
                load(g + 1, 1 - b)

            wait_in(b)

            @pl.when(g >= 2)
            def _():
                wait_out(b)

            for r in range(_G):
                o = (b * _G + r) * _L

                @plsc.parallel_loop(0, _L // _LANES, unroll=8)
                def _blk(k):
                    p = k * _LANES
                    v = in_v[pl.ds(o + p, _LANES)]
                    iv = idx_v[pl.ds(p, _LANES)] + o
                    plsc.store_scatter(out_v, [iv], v)

            store(g, b)

    wait_out(0)
    wait_out(1)


def _sc_permute(xf, idx):
    mesh = plsc.VectorSubcoreMesh(core_axis_name="c", subcore_axis_name="s")
    return pl.kernel(
        _sc_body,
        out_type=jax.ShapeDtypeStruct((_ROWS, _L), jnp.float32),
        mesh=mesh,
        scratch_types=[
            pltpu.VMEM((_L,), jnp.int32),
            pltpu.VMEM((2 * _GL,), jnp.float32),
            pltpu.VMEM((2 * _GL,), jnp.float32),
            pltpu.SemaphoreType.DMA((2,)),
            pltpu.SemaphoreType.DMA((2,)),
        ],
        compiler_params=pltpu.CompilerParams(needs_layout_passes=False),
    )(xf, idx)


_CH = 2048


def _tc_tr_body(z_ref, w_ref):
    w_ref[0] = jnp.swapaxes(z_ref[0], 0, 1)


def _tc_transpose(z3):
    return pl.pallas_call(
        _tc_tr_body,
        out_shape=jax.ShapeDtypeStruct((_B, _L, _C), jnp.float32),
        grid=(_B, _L // _CH),
        in_specs=[pl.BlockSpec((1, _C, _CH), lambda b, k: (b, 0, k))],
        out_specs=pl.BlockSpec((1, _CH, _C), lambda b, k: (b, k, 0)),
        compiler_params=pltpu.CompilerParams(
            dimension_semantics=("parallel", "parallel")),
    )(z3)


def kernel(x):
    xf = x.reshape(_ROWS, _L)
    idx = jnp.asarray(_IDX_NP)
    z = _sc_permute(xf, idx)
    w = _tc_transpose(z.reshape(_B, _C, _L))
    y = w.reshape(_B, _S, _S, _C)
    return y.transpose(0, 3, 1, 2)


# speedup vs baseline: 1.4516x; 1.0772x over previous
import numpy as np
import jax
import jax.numpy as jnp
from jax import lax
from jax.experimental import pallas as pl
from jax.experimental.pallas import tpu as pltpu
from jax.experimental.pallas import tpu_sc as plsc

_B, _C, _L = 16, 192, 4096
_S = 64
_ROWS = _B * _C
_NC, _NS = 2, 16
_NW = _NC * _NS
_RPW = _ROWS // _NW
_LANES = 16
_G = 6
_GL = _G * _L
_NG = _RPW // _G


def _morton_idx(l: int) -> np.ndarray:
    s = int(np.sqrt(l))
    ij = np.arange(l, dtype=np.int64)
    i = np.zeros(l, dtype=np.int64)
    j = np.zeros(l, dtype=np.int64)
    for t in range(int(l).bit_length() // 2 + 1):
        i += ((ij >> (2 * t + 1)) & 1) << t
        j += ((ij >> (2 * t)) & 1) << t
    return (i * s + j).astype(np.int32)


_IDX_NP = _morton_idx(_L)


def _sc_body(x_hbm, idx_hbm, out_hbm, idx_v, in_v, out_v, in_sem, out_sem):
    wid = lax.axis_index("s") * _NC + lax.axis_index("c")
    row0 = wid * _RPW
    pltpu.sync_copy(idx_hbm, idx_v)

    def load(g, b):
        for r in range(_G):
            pltpu.async_copy(x_hbm.at[row0 + g * _G + r],
                             in_v.at[pl.ds((b * _G + r) * _L, _L)],
                             in_sem.at[b])

    def wait_in(b):
        for r in range(_G):
            pltpu.make_async_copy(x_hbm.at[0],
                                  in_v.at[pl.ds((b * _G + r) * _L, _L)],
                                  in_sem.at[b]).wait()

    def store(g, b):
        for r in range(_G):
            pltpu.async_copy(out_v.at[pl.ds((b * _G + r) * _L, _L)],
                             out_hbm.at[row0 + g * _G + r], out_sem.at[b])

    def wait_out(b):
        for r in range(_G):
            pltpu.make_async_copy(out_v.at[pl.ds((b * _G + r) * _L, _L)],
                                  out_hbm.at[0], out_sem.at[b]).wait()

    load(0, 0)

    @pl.loop(0, _NG, step=2)
    def _grp(g0):
        for b in range(2):
            g = g0 + b

            @pl.when(g + 1 < _NG)
            def _():
                load(g + 1, 1 - b)

            wait_in(b)

            @pl.when(g >= 2)
            def _():
                wait_out(b)

            for r in range(_G):
                o = (b * _G + r) * _L

                @plsc.parallel_loop(0, _L // _LANES, unroll=8)
                def _blk(k):
                    p = k * _LANES
                    v = in_v[pl.ds(o + p, _LANES)]
                    iv = idx_v[pl.ds(p, _LANES)] + o
                    plsc.store_scatter(out_v, [iv], v)

            store(g, b)

    wait_out(0)
    wait_out(1)


def _sc_permute(xf, idx):
    mesh = plsc.VectorSubcoreMesh(core_axis_name="c", subcore_axis_name="s")
    return pl.kernel(
        _sc_body,
        out_type=jax.ShapeDtypeStruct((_ROWS, _L), jnp.float32),
        mesh=mesh,
        scratch_types=[
            pltpu.VMEM((_L,), jnp.int32),
            pltpu.VMEM((2 * _GL,), jnp.float32),
            pltpu.VMEM((2 * _GL,), jnp.float32),
            pltpu.SemaphoreType.DMA((2,)),
            pltpu.SemaphoreType.DMA((2,)),
        ],
        compiler_params=pltpu.CompilerParams(needs_layout_passes=False),
    )(xf, idx)


_CH = 4096


def _tc_tr_body(z_ref, w_ref):
    w_ref[0] = jnp.swapaxes(z_ref[0], 0, 1)


def _tc_transpose(z3):
    return pl.pallas_call(
        _tc_tr_body,
        out_shape=jax.ShapeDtypeStruct((_B, _L, _C), jnp.float32),
        grid=(_B, _L // _CH),
        in_specs=[pl.BlockSpec((1, _C, _CH), lambda b, k: (b, 0, k))],
        out_specs=pl.BlockSpec((1, _CH, _C), lambda b, k: (b, k, 0)),
        compiler_params=pltpu.CompilerParams(
            dimension_semantics=("parallel", "parallel")),
    )(z3)


def kernel(x):
    xf = x.reshape(_ROWS, _L)
    idx = jnp.asarray(_IDX_NP)
    z = _sc_permute(xf, idx)
    w = _tc_transpose(z.reshape(_B, _C, _L))
    y = w.reshape(_B, _S, _S, _C)
    return y.transpose(0, 3, 1, 2)
